# baseline (speedup 1.0000x reference)
import functools

import jax
import jax.numpy as jnp
from jax import lax
from jax.experimental import pallas as pl
from jax.experimental.pallas import tpu as pltpu
from jax.experimental.pallas import tpu_sc as plsc

B = 4096
HIST = 200
D = 64
NC = 2
NS = 16
NW = NC * NS
BPW = B // NW
G = 16
NTCF = HIST // 16
CHUNKS = ((0, 104), (104, 96))
LAM = 0.001


def _sc_body(item_hbm, user_hbm, hist_hbm, ts_hbm, uid_hbm, iid_hbm,
             hacc_hbm, ws_hbm, uemb_hbm, iemb_hbm,
             hist_v, ts_v, w_v, rows0, rows1, rows2, rows3, acc_v, ws_v,
             uid_v, iid_v, uemb_v, iemb_v, sem0, sem1, sem2, sem3, sem_e):
    wid = lax.axis_index("s") * NC + lax.axis_index("c")
    base = wid * BPW
    rows = (rows0, rows1, rows2, rows3)
    sems = (sem0, sem1, sem2, sem3)

    pltpu.sync_copy(uid_hbm.at[pl.ds(base, BPW)], uid_v)
    pltpu.sync_copy(iid_hbm.at[pl.ds(base, BPW)], iid_v)
    cu = pltpu.async_copy(user_hbm.at[uid_v], uemb_v, sem_e)
    ci = pltpu.async_copy(item_hbm.at[iid_v], iemb_v, sem_e)

    pltpu.sync_copy(hist_hbm.at[pl.ds(base, BPW)], hist_v)

    tail_keep = lax.broadcasted_iota(jnp.int32, (16,), 0) >= 8

    def fire(b, k):
        for off, ln in CHUNKS:
            pltpu.async_copy(
                item_hbm.at[hist_v.at[b, pl.ds(off, ln)]],
                rows[k].at[pl.ds(off, ln)], sems[k])

    def wait_g(b, k):
        for off, ln in CHUNKS:
            pltpu.make_async_copy(
                item_hbm.at[hist_v.at[b, pl.ds(off, ln)]],
                rows[k].at[pl.ds(off, ln)], sems[k]).wait()

    def compute_w(tsl, b):
        ws = jnp.zeros((16,), jnp.float32)
        for tc in range(NTCF + 1):
            off = tc * 16 if tc < NTCF else HIST - 16
            ts = ts_v[tsl, pl.ds(off, 16)]
            h = hist_v[b, pl.ds(off, 16)]
            m = jnp.where(h > 0, 1.0, 0.0).astype(jnp.float32)
            wch = jnp.exp(ts * (-LAM)) * m
            if tc == NTCF:
                wch = jnp.where(tail_keep, wch, 0.0)
            w_v[pl.ds(tc * 16, 16)] = wch
            ws = ws + wch
        ws_v[pl.ds(b * 16, 16)] = ws

    def fma(b, k):
        rk = rows[k]

        def halves(iv):
            lo = plsc.bitcast(iv << 16, jnp.float32)
            hi = plsc.bitcast(iv & jnp.int32(-65536), jnp.float32)
            return lo, hi

        def step(t, w, acc):
            a0, a1, a2, a3 = acc
            e0, o0 = halves(rk[t, pl.ds(0, 16)])
            e1, o1 = halves(rk[t, pl.ds(16, 16)])
            return (a0 + w * e0, a1 + w * o0, a2 + w * e1, a3 + w * o1)

        def tbody(tc, acc):
            wch = w_v[pl.ds(tc * 16, 16)]
            for j in range(16):
                acc = step(tc * 16 + j, wch[j], acc)
            return acc

        z = jnp.zeros((16,), jnp.float32)
        acc = lax.fori_loop(0, NTCF, tbody, (z, z, z, z))
        wch = w_v[pl.ds(NTCF * 16, 16)]
        for j in range(8, 16):
            acc = step(HIST - 16 + j, wch[j], acc)
        a0, a1, a2, a3 = acc
        acc_v[pl.ds(b * 64, 16)] = a0
        acc_v[pl.ds(b * 64 + 16, 16)] = a1
        acc_v[pl.ds(b * 64 + 32, 16)] = a2
        acc_v[pl.ds(b * 64 + 48, 16)] = a3

    for k in range(4):
        fire(k, k)

    def body4(i, _):
        for k in range(4):
            b = 4 * i + k
            if k == 0:
                @pl.when(lax.rem(i, 4) == 0)
                def _():
                    pltpu.sync_copy(ts_hbm.at[pl.ds(base + 4 * i, G)], ts_v)
            tsl = 4 * lax.rem(i, 4) + k
            compute_w(tsl, b)
            wait_g(b, k)
            fma(b, k)

            @pl.when(i < BPW // 4 - 1)
            def _():
                fire(b + 4, k)
        return 0

    lax.fori_loop(0, BPW // 4, body4, 0)

    cu.wait()
    ci.wait()
    pltpu.sync_copy(acc_v, hacc_hbm.at[pl.ds(base * 64, BPW * 64)])
    pltpu.sync_copy(ws_v, ws_hbm.at[pl.ds(base * 16, BPW * 16)])
    pltpu.sync_copy(uemb_v, uemb_hbm.at[pl.ds(base, BPW)])
    pltpu.sync_copy(iemb_v, iemb_hbm.at[pl.ds(base, BPW)])


def _sc_pool(item_table, user_table, hist2d, ts2d, uid, iid):
    mesh = plsc.VectorSubcoreMesh(core_axis_name="c", subcore_axis_name="s",
                                  num_cores=NC, num_subcores=NS)
    f32 = jnp.float32
    run = pl.kernel(
        _sc_body,
        out_type=[
            jax.ShapeDtypeStruct((B * D,), f32),
            jax.ShapeDtypeStruct((B * 16,), f32),
            jax.ShapeDtypeStruct((B, D // 2), jnp.int32),
            jax.ShapeDtypeStruct((B, D // 2), jnp.int32),
        ],
        mesh=mesh,
        compiler_params=pltpu.CompilerParams(needs_layout_passes=False,
                                             use_tc_tiling_on_sc=False),
        scratch_types=[
            pltpu.VMEM((BPW, HIST), jnp.int32),
            pltpu.VMEM((G, HIST), f32),
            pltpu.VMEM(((NTCF + 1) * 16,), f32),
            pltpu.VMEM((HIST, D // 2), jnp.int32),
            pltpu.VMEM((HIST, D // 2), jnp.int32),
            pltpu.VMEM((HIST, D // 2), jnp.int32),
            pltpu.VMEM((HIST, D // 2), jnp.int32),
            pltpu.VMEM((BPW * D,), f32),
            pltpu.VMEM((BPW * 16,), f32),
            pltpu.VMEM((BPW,), jnp.int32),
            pltpu.VMEM((BPW,), jnp.int32),
            pltpu.VMEM((BPW, D // 2), jnp.int32),
            pltpu.VMEM((BPW, D // 2), jnp.int32),
            pltpu.SemaphoreType.DMA,
            pltpu.SemaphoreType.DMA,
            pltpu.SemaphoreType.DMA,
            pltpu.SemaphoreType.DMA,
            pltpu.SemaphoreType.DMA,
        ],
    )
    return run(item_table, user_table, hist2d, ts2d, uid, iid)


BB = 1024


def _pre_body(tg, ig, uar, act, ry, iar, rev, gtab, ucW, ucb, icW, icb,
              ugp, igp, ucont, icont):
    f32 = jnp.float32
    iota = lax.broadcasted_iota(jnp.int32, (1, 64), 1)

    def genre_pool(g, ng):
        C = jnp.zeros((BB, 64), f32)
        cnt = jnp.zeros((BB, 1), f32)
        for j in range(ng):
            gj = g[:, j:j + 1]
            pos = gj > 0
            C = C + jnp.where((gj == iota) & pos, 1.0, 0.0)
            cnt = cnt + jnp.where(pos, 1.0, 0.0)
        pooled = jnp.dot(C, gtab[...], preferred_element_type=f32)
        return pooled / (cnt + 1e-8)

    ugp[...] = genre_pool(tg[...], 8)
    igp[...] = genre_pool(ig[...], 4)
    ucont[...] = jnp.maximum(
        uar[...] * ucW[0:1, :] + act[...] * ucW[1:2, :] + ucb[...], 0.0)
    icont[...] = jnp.maximum(
        ry[...] * icW[0:1, :] + iar[...] * icW[1:2, :]
        + rev[...] * icW[2:3, :] + icb[...], 0.0)


def _tower_body(uemb, hacc, ws16, iemb, ugp, igp, ucont, icont,
                uW1, ub1, ulg, ulb, uW2, ub2,
                iW1, ib1, ilg, ilb, iW2, ib2, uo, io):
    f32 = jnp.float32

    def unpack(iv):
        ev = lax.bitcast_convert_type(lax.shift_left(iv, 16), f32)
        od = lax.bitcast_convert_type(
            lax.bitwise_and(iv, jnp.int32(-65536)), f32)
        return ev, od

    ue, uo_ = unpack(uemb[...])
    ie, io_ = unpack(iemb[...])
    hacc2 = hacc[...]
    ws2 = ws16[...]

    def tower(cat, W1, b1, lg, lb, W2, b2):
        h = jnp.dot(cat, W1[...], preferred_element_type=f32) + b1[...]
        m = jnp.mean(h, axis=-1, keepdims=True)
        v = jnp.mean((h - m) ** 2, axis=-1, keepdims=True)
        h = (h - m) / jnp.sqrt(v + 1e-5) * lg[...] + lb[...]
        h = jnp.maximum(h, 0.0)
        o = jnp.dot(h, W2[...], preferred_element_type=f32) + b2[...]
        n = jnp.sqrt(jnp.sum(o * o, axis=1, keepdims=True))
        return o / jnp.maximum(n, 1e-12)

    hist_pooled = hacc2 / (jnp.sum(ws2, axis=1, keepdims=True) + 1e-8)
    u_cat = jnp.concatenate(
        [ue, uo_, hist_pooled, ugp[...], ucont[...]], axis=1)
    uo[...] = tower(u_cat, uW1, ub1, ulg, ulb, uW2, ub2)
    i_cat = jnp.concatenate(
        [ie, io_, igp[...], icont[...]], axis=1)
    io[...] = tower(i_cat, iW1, ib1, ilg, ilb, iW2, ib2)


def _bspec(x):
    return pl.BlockSpec((BB,) + x.shape[1:],
                        lambda i: (i,) + (0,) * (x.ndim - 1))


def _fspec(x):
    return pl.BlockSpec(x.shape, lambda i: (0,) * x.ndim)


def _tc_pre(tg, ig, uar, act, ry, iar, rev, gtab, ucW, ucb, icW, icb):
    f32 = jnp.float32
    batch_args = (tg, ig, uar, act, ry, iar, rev)
    full_args = (gtab, ucW, ucb, icW, icb)
    in_specs = [_bspec(x) for x in batch_args] + [_fspec(x) for x in full_args]
    return pl.pallas_call(
        _pre_body,
        grid=(B // BB,),
        in_specs=in_specs,
        out_specs=[pl.BlockSpec((BB, D), lambda i: (i, 0))] * 4,
        out_shape=[jax.ShapeDtypeStruct((B, D), f32)] * 4,
    )(*batch_args, *full_args)


def _tc_towers(uemb, hacc, ws16, iemb, ugp, igp, ucont, icont,
               uW1, ub1, ulg, ulb, uW2, ub2,
               iW1, ib1, ilg, ilb, iW2, ib2):
    f32 = jnp.float32
    batch_args = (uemb, hacc, ws16, iemb, ugp, igp, ucont, icont)
    full_args = (uW1, ub1, ulg, ulb, uW2, ub2, iW1, ib1, ilg, ilb, iW2, ib2)
    in_specs = [_bspec(x) for x in batch_args] + [_fspec(x) for x in full_args]
    return pl.pallas_call(
        _tower_body,
        grid=(B // BB,),
        in_specs=in_specs,
        out_specs=[pl.BlockSpec((BB, D), lambda i: (i, 0))] * 2,
        out_shape=[jax.ShapeDtypeStruct((B, D), f32)] * 2,
    )(*batch_args, *full_args)


def kernel(user_id, history, history_ts_diff, top_genres, user_avg_rating,
           activity, item_id, tmdb_genres, release_year, item_avg_rating,
           revenue, user_table, item_table, genre_table, u_cont_W, u_cont_b,
           u_mlp_W1, u_mlp_b1, u_ln_g, u_ln_b, u_mlp_W2, u_mlp_b2, i_cont_W,
           i_cont_b, i_mlp_W1, i_mlp_b1, i_ln_g, i_ln_b, i_mlp_W2, i_mlp_b2):
    i32, f32 = jnp.int32, jnp.float32

    def pack(tbl):
        bf = tbl.astype(jnp.bfloat16).reshape(-1, D // 2, 2)
        return jax.lax.bitcast_convert_type(bf, i32)

    hacc, ws16, uemb, iemb = _sc_pool(
        pack(item_table), pack(user_table),
        history.astype(i32), history_ts_diff.astype(f32),
        user_id.astype(i32), item_id.astype(i32))
    hacc = hacc.reshape(B, D)
    ws16 = ws16.reshape(B, 16)

    perm = (list(range(0, 32, 2)) + list(range(1, 32, 2))
            + list(range(32, 64, 2)) + list(range(33, 64, 2)))
    perm64 = list(range(0, 64, 2)) + list(range(1, 64, 2))
    uW1 = jnp.concatenate(
        [u_mlp_W1[:D][jnp.array(perm64)],
         u_mlp_W1[D:2 * D][jnp.array(perm)], u_mlp_W1[2 * D:]], axis=0)
    iW1 = jnp.concatenate(
        [i_mlp_W1[:D][jnp.array(perm64)], i_mlp_W1[D:]], axis=0)

    gtab = jnp.concatenate([genre_table, jnp.zeros((31, D), f32)], axis=0)
    ucW = jnp.concatenate([u_cont_W, jnp.zeros((6, D), f32)], axis=0)
    icW = jnp.concatenate([i_cont_W, jnp.zeros((5, D), f32)], axis=0)
    col = lambda x: x.astype(f32).reshape(B, 1)
    row = lambda x: x.reshape(1, -1)

    ugp, igp, ucont, icont = _tc_pre(
        top_genres.astype(i32), tmdb_genres.astype(i32),
        col(user_avg_rating), col(activity), col(release_year),
        col(item_avg_rating), col(revenue),
        gtab, ucW, row(u_cont_b), icW, row(i_cont_b))

    u_out, i_out = _tc_towers(
        uemb, hacc, ws16, iemb, ugp, igp, ucont, icont,
        uW1, row(u_mlp_b1), row(u_ln_g), row(u_ln_b), u_mlp_W2,
        row(u_mlp_b2),
        iW1, row(i_mlp_b1), row(i_ln_g), row(i_ln_b), i_mlp_W2,
        row(i_mlp_b2))
    return (u_out, i_out)


# speedup vs baseline: 1.6346x; 1.6346x over previous
import functools

import jax
import jax.numpy as jnp
from jax import lax
from jax.experimental import pallas as pl
from jax.experimental.pallas import tpu as pltpu
from jax.experimental.pallas import tpu_sc as plsc

B = 4096
HIST = 200
D = 64
NC = 2
NS = 16
NW = NC * NS
BPW = B // NW
G = 16
NTCF = HIST // 16
CHUNKS = ((0, 104), (104, 96))
LAM = 0.001


def _sc_body(item_hbm, user_hbm, hist_hbm, ts_hbm, uid_hbm, iid_hbm,
             hacc_hbm, ws_hbm, uemb_hbm, iemb_hbm,
             hist_v, ts_v, w_v, rows0, rows1, rows2, rows3, acc_v, ws_v,
             uid_v, iid_v, uemb_v, iemb_v, sem0, sem1, sem2, sem3, sem_e):
    wid = lax.axis_index("s") * NC + lax.axis_index("c")
    base = wid * BPW
    rows = (rows0, rows1, rows2, rows3)
    sems = (sem0, sem1, sem2, sem3)

    pltpu.sync_copy(uid_hbm.at[pl.ds(base, BPW)], uid_v)
    pltpu.sync_copy(iid_hbm.at[pl.ds(base, BPW)], iid_v)
    cu = pltpu.async_copy(user_hbm.at[uid_v], uemb_v, sem_e)
    ci = pltpu.async_copy(item_hbm.at[iid_v], iemb_v, sem_e)

    pltpu.sync_copy(hist_hbm.at[pl.ds(base, BPW)], hist_v)

    tail_keep = lax.broadcasted_iota(jnp.int32, (16,), 0) >= 8

    def fire(b, k):
        for off, ln in CHUNKS:
            pltpu.async_copy(
                item_hbm.at[hist_v.at[b, pl.ds(off, ln)]],
                rows[k].at[pl.ds(off, ln)], sems[k])

    def wait_g(b, k):
        for off, ln in CHUNKS:
            pltpu.make_async_copy(
                item_hbm.at[hist_v.at[b, pl.ds(off, ln)]],
                rows[k].at[pl.ds(off, ln)], sems[k]).wait()

    def compute_w(tsl, b):
        ws = jnp.zeros((16,), jnp.float32)
        for tc in range(NTCF + 1):
            off = tc * 16 if tc < NTCF else HIST - 16
            ts = ts_v[tsl, pl.ds(off, 16)]
            h = hist_v[b, pl.ds(off, 16)]
            m = jnp.where(h > 0, 1.0, 0.0).astype(jnp.float32)
            wch = jnp.exp(ts * (-LAM)) * m
            if tc == NTCF:
                wch = jnp.where(tail_keep, wch, 0.0)
            w_v[pl.ds(tc * 16, 16)] = wch
            ws = ws + wch
        ws_v[pl.ds(b * 16, 16)] = ws

    def fma(b, k):
        rk = rows[k]

        def halves(v):
            iv = plsc.bitcast(v, jnp.int32)
            lo = plsc.bitcast(iv << 16, jnp.float32)
            hi = plsc.bitcast(iv & jnp.int32(-65536), jnp.float32)
            return lo, hi

        def step(t, w, acc):
            a0, a1, a2, a3 = acc
            e0, o0 = halves(rk[t, pl.ds(0, 32)])
            e1, o1 = halves(rk[t, pl.ds(32, 32)])
            return (a0 + w * e0, a1 + w * o0, a2 + w * e1, a3 + w * o1)

        def tbody(tc, acc):
            wch = w_v[pl.ds(tc * 16, 16)]
            for j in range(16):
                acc = step(tc * 16 + j, wch[j], acc)
            return acc

        z = jnp.zeros((16,), jnp.float32)
        acc = lax.fori_loop(0, NTCF, tbody, (z, z, z, z))
        wch = w_v[pl.ds(NTCF * 16, 16)]
        for j in range(8, 16):
            acc = step(HIST - 16 + j, wch[j], acc)
        a0, a1, a2, a3 = acc
        acc_v[pl.ds(b * 64, 16)] = a0
        acc_v[pl.ds(b * 64 + 16, 16)] = a1
        acc_v[pl.ds(b * 64 + 32, 16)] = a2
        acc_v[pl.ds(b * 64 + 48, 16)] = a3

    for k in range(4):
        fire(k, k)

    def body4(i, _):
        for k in range(4):
            b = 4 * i + k
            if k == 0:
                @pl.when(lax.rem(i, 4) == 0)
                def _():
                    pltpu.sync_copy(ts_hbm.at[pl.ds(base + 4 * i, G)], ts_v)
            tsl = 4 * lax.rem(i, 4) + k
            compute_w(tsl, b)
            wait_g(b, k)
            fma(b, k)

            @pl.when(i < BPW // 4 - 1)
            def _():
                fire(b + 4, k)
        return 0

    lax.fori_loop(0, BPW // 4, body4, 0)

    cu.wait()
    ci.wait()
    pltpu.sync_copy(acc_v, hacc_hbm.at[pl.ds(base * 64, BPW * 64)])
    pltpu.sync_copy(ws_v, ws_hbm.at[pl.ds(base * 16, BPW * 16)])
    pltpu.sync_copy(uemb_v, uemb_hbm.at[pl.ds(base, BPW)])
    pltpu.sync_copy(iemb_v, iemb_hbm.at[pl.ds(base, BPW)])


def _sc_pool(item_table, user_table, hist2d, ts2d, uid, iid):
    mesh = plsc.VectorSubcoreMesh(core_axis_name="c", subcore_axis_name="s",
                                  num_cores=NC, num_subcores=NS)
    f32 = jnp.float32
    run = pl.kernel(
        _sc_body,
        out_type=[
            jax.ShapeDtypeStruct((B * D,), f32),
            jax.ShapeDtypeStruct((B * 16,), f32),
            jax.ShapeDtypeStruct((B, D), jnp.bfloat16),
            jax.ShapeDtypeStruct((B, D), jnp.bfloat16),
        ],
        mesh=mesh,
        compiler_params=pltpu.CompilerParams(needs_layout_passes=False,
                                             use_tc_tiling_on_sc=False),
        scratch_types=[
            pltpu.VMEM((BPW, HIST), jnp.int32),
            pltpu.VMEM((G, HIST), f32),
            pltpu.VMEM(((NTCF + 1) * 16,), f32),
            pltpu.VMEM((HIST, D), jnp.bfloat16),
            pltpu.VMEM((HIST, D), jnp.bfloat16),
            pltpu.VMEM((HIST, D), jnp.bfloat16),
            pltpu.VMEM((HIST, D), jnp.bfloat16),
            pltpu.VMEM((BPW * D,), f32),
            pltpu.VMEM((BPW * 16,), f32),
            pltpu.VMEM((BPW,), jnp.int32),
            pltpu.VMEM((BPW,), jnp.int32),
            pltpu.VMEM((BPW, D), jnp.bfloat16),
            pltpu.VMEM((BPW, D), jnp.bfloat16),
            pltpu.SemaphoreType.DMA,
            pltpu.SemaphoreType.DMA,
            pltpu.SemaphoreType.DMA,
            pltpu.SemaphoreType.DMA,
            pltpu.SemaphoreType.DMA,
        ],
    )
    return run(item_table, user_table, hist2d, ts2d, uid, iid)


BB = 1024


def _pre_body(tg, ig, uar, act, ry, iar, rev, gtab, ucW, ucb, icW, icb,
              ugp, igp, ucont, icont):
    f32 = jnp.float32
    iota = lax.broadcasted_iota(jnp.int32, (1, 64), 1)

    def genre_pool(g, ng):
        C = jnp.zeros((BB, 64), f32)
        cnt = jnp.zeros((BB, 1), f32)
        for j in range(ng):
            gj = g[:, j:j + 1]
            pos = gj > 0
            C = C + jnp.where((gj == iota) & pos, 1.0, 0.0)
            cnt = cnt + jnp.where(pos, 1.0, 0.0)
        pooled = jnp.dot(C, gtab[...], preferred_element_type=f32)
        return pooled / (cnt + 1e-8)

    ugp[...] = genre_pool(tg[...], 8)
    igp[...] = genre_pool(ig[...], 4)
    ucont[...] = jnp.maximum(
        uar[...] * ucW[0:1, :] + act[...] * ucW[1:2, :] + ucb[...], 0.0)
    icont[...] = jnp.maximum(
        ry[...] * icW[0:1, :] + iar[...] * icW[1:2, :]
        + rev[...] * icW[2:3, :] + icb[...], 0.0)


def _tower_body(uemb, hacc, ws16, iemb, ugp, igp, ucont, icont,
                uW1, ub1, ulg, ulb, uW2, ub2,
                iW1, ib1, ilg, ilb, iW2, ib2, uo, io):
    f32 = jnp.float32
    uemb2 = uemb[...].astype(f32)
    iemb2 = iemb[...].astype(f32)
    hacc2 = hacc[...]
    ws2 = ws16[...]

    def tower(cat, W1, b1, lg, lb, W2, b2):
        h = jnp.dot(cat, W1[...], preferred_element_type=f32) + b1[...]
        m = jnp.mean(h, axis=-1, keepdims=True)
        v = jnp.mean((h - m) ** 2, axis=-1, keepdims=True)
        h = (h - m) / jnp.sqrt(v + 1e-5) * lg[...] + lb[...]
        h = jnp.maximum(h, 0.0)
        o = jnp.dot(h, W2[...], preferred_element_type=f32) + b2[...]
        n = jnp.sqrt(jnp.sum(o * o, axis=1, keepdims=True))
        return o / jnp.maximum(n, 1e-12)

    hist_pooled = hacc2 / (jnp.sum(ws2, axis=1, keepdims=True) + 1e-8)
    u_cat = jnp.concatenate(
        [uemb2, hist_pooled, ugp[...], ucont[...]], axis=1)
    uo[...] = tower(u_cat, uW1, ub1, ulg, ulb, uW2, ub2)
    i_cat = jnp.concatenate(
        [iemb2, igp[...], icont[...]], axis=1)
    io[...] = tower(i_cat, iW1, ib1, ilg, ilb, iW2, ib2)


def _bspec(x):
    return pl.BlockSpec((BB,) + x.shape[1:],
                        lambda i: (i,) + (0,) * (x.ndim - 1))


def _fspec(x):
    return pl.BlockSpec(x.shape, lambda i: (0,) * x.ndim)


def _tc_pre(tg, ig, uar, act, ry, iar, rev, gtab, ucW, ucb, icW, icb):
    f32 = jnp.float32
    batch_args = (tg, ig, uar, act, ry, iar, rev)
    full_args = (gtab, ucW, ucb, icW, icb)
    in_specs = [_bspec(x) for x in batch_args] + [_fspec(x) for x in full_args]
    return pl.pallas_call(
        _pre_body,
        grid=(B // BB,),
        in_specs=in_specs,
        out_specs=[pl.BlockSpec((BB, D), lambda i: (i, 0))] * 4,
        out_shape=[jax.ShapeDtypeStruct((B, D), f32)] * 4,
    )(*batch_args, *full_args)


def _tc_towers(uemb, hacc, ws16, iemb, ugp, igp, ucont, icont,
               uW1, ub1, ulg, ulb, uW2, ub2,
               iW1, ib1, ilg, ilb, iW2, ib2):
    f32 = jnp.float32
    batch_args = (uemb, hacc, ws16, iemb, ugp, igp, ucont, icont)
    full_args = (uW1, ub1, ulg, ulb, uW2, ub2, iW1, ib1, ilg, ilb, iW2, ib2)
    in_specs = [_bspec(x) for x in batch_args] + [_fspec(x) for x in full_args]
    return pl.pallas_call(
        _tower_body,
        grid=(B // BB,),
        in_specs=in_specs,
        out_specs=[pl.BlockSpec((BB, D), lambda i: (i, 0))] * 2,
        out_shape=[jax.ShapeDtypeStruct((B, D), f32)] * 2,
    )(*batch_args, *full_args)


def kernel(user_id, history, history_ts_diff, top_genres, user_avg_rating,
           activity, item_id, tmdb_genres, release_year, item_avg_rating,
           revenue, user_table, item_table, genre_table, u_cont_W, u_cont_b,
           u_mlp_W1, u_mlp_b1, u_ln_g, u_ln_b, u_mlp_W2, u_mlp_b2, i_cont_W,
           i_cont_b, i_mlp_W1, i_mlp_b1, i_ln_g, i_ln_b, i_mlp_W2, i_mlp_b2):
    i32, f32 = jnp.int32, jnp.float32
    hacc, ws16, uemb, iemb = _sc_pool(
        item_table.astype(jnp.bfloat16), user_table.astype(jnp.bfloat16),
        history.astype(i32), history_ts_diff.astype(f32),
        user_id.astype(i32), item_id.astype(i32))
    hacc = hacc.reshape(B, D)
    ws16 = ws16.reshape(B, 16)

    perm = (list(range(0, 32, 2)) + list(range(1, 32, 2))
            + list(range(32, 64, 2)) + list(range(33, 64, 2)))
    uW1 = jnp.concatenate(
        [u_mlp_W1[:D], u_mlp_W1[D:2 * D][jnp.array(perm)], u_mlp_W1[2 * D:]],
        axis=0)

    gtab = jnp.concatenate([genre_table, jnp.zeros((31, D), f32)], axis=0)
    ucW = jnp.concatenate([u_cont_W, jnp.zeros((6, D), f32)], axis=0)
    icW = jnp.concatenate([i_cont_W, jnp.zeros((5, D), f32)], axis=0)
    col = lambda x: x.astype(f32).reshape(B, 1)
    row = lambda x: x.reshape(1, -1)

    ugp, igp, ucont, icont = _tc_pre(
        top_genres.astype(i32), tmdb_genres.astype(i32),
        col(user_avg_rating), col(activity), col(release_year),
        col(item_avg_rating), col(revenue),
        gtab, ucW, row(u_cont_b), icW, row(i_cont_b))

    u_out, i_out = _tc_towers(
        uemb, hacc, ws16, iemb, ugp, igp, ucont, icont,
        uW1, row(u_mlp_b1), row(u_ln_g), row(u_ln_b), u_mlp_W2,
        row(u_mlp_b2),
        i_mlp_W1, row(i_mlp_b1), row(i_ln_g), row(i_ln_b), i_mlp_W2,
        row(i_mlp_b2))
    return (u_out, i_out)
